# baseline (speedup 1.0000x reference)
import functools

import jax
import jax.numpy as jnp
from jax.experimental import pallas as pl
from jax.experimental.pallas import tpu as pltpu

B, S, DIN, E, G, C = 16, 4096, 8, 64, 128, 8

_dott = functools.partial(jnp.dot, preferred_element_type=jnp.float32)


def _stats_kernel(len_ref, xf_ref, lenrow_ref, stats_ref):
    lane = jax.lax.broadcasted_iota(jnp.int32, (1, S), 1)
    xf = xf_ref[...]
    maskf = lane < lenrow_ref[...]
    xm = jnp.where(maskf, xf, 0.0)
    s1 = jnp.sum(xm, axis=1, keepdims=True)
    s2 = jnp.sum(xm * xm, axis=1, keepdims=True)
    s12 = jnp.concatenate([s1, s2], axis=1)
    sums = s12[0:DIN, :]
    for i in range(1, B):
        sums = sums + s12[i * DIN:(i + 1) * DIN, :]
    cnt = jax.lax.fori_loop(0, B, lambda i, c: c + len_ref[i], 0)
    cntf = jnp.maximum(cnt.astype(jnp.float32), 1.0)
    mean = sums[:, 0:1] / cntf
    var = sums[:, 1:2] / cntf - mean * mean
    scale = 1.0 / jnp.sqrt(var + 1e-6)
    stats_ref[...] = jnp.concatenate([mean, scale], axis=1)


def _fused_kernel(len_ref, xb_ref, stats_ref, w1_ref, b1_ref, w2_ref,
                  b2_ref, wg_ref, bg_ref, wseg_ref, bseg_ref,
                  logits_ref, labels_ref, scores_ref, cat_ref):
    b = pl.program_id(0)
    lane = jax.lax.broadcasted_iota(jnp.int32, (1, S), 1)
    ln = len_ref[b]
    mask = lane < ln
    xb = xb_ref[0]
    mean = stats_ref[:, 0:1]
    scale = stats_ref[:, 1:2]
    xn = jnp.where(mask, (xb - mean) * scale, 0.0)

    h = jax.nn.relu(_dott(w1_ref[...], xn) + b1_ref[...])
    h = jax.nn.relu(_dott(w2_ref[...], h) + b2_ref[...])
    h = jnp.where(mask, h, 0.0)

    g = jax.nn.relu(_dott(wg_ref[...], h) + bg_ref[...])
    g = jnp.where(mask, g, -jnp.inf)
    gfv = jnp.max(g, axis=1, keepdims=True)

    gfv_bc = jnp.where(mask, gfv, 0.0)
    cat = jnp.concatenate([h, gfv_bc], axis=0)
    cat_ref[0] = cat

    logits = _dott(wseg_ref[...], cat) + bseg_ref[...]
    logits = jnp.where(mask, logits, 0.0)
    logits_ref[0] = logits

    m = jnp.max(logits, axis=0, keepdims=True)
    e = jnp.exp(logits - m)
    p = e / jnp.sum(e, axis=0, keepdims=True)
    score = jnp.max(p, axis=0, keepdims=True)
    cidx = jax.lax.broadcasted_iota(jnp.int32, (C, S), 0)
    lab = jnp.min(jnp.where(p >= score, cidx, C), axis=0, keepdims=True)
    lab = jnp.where(jnp.isnan(score), -1, lab).astype(jnp.int32)
    labels_ref[0] = lab
    scores_ref[0] = score


def kernel(x, lengths, W1, b1, W2, b2, Wg, bg, Wseg, bseg):
    lengths = lengths.astype(jnp.int32)
    xf = jnp.transpose(x, (0, 2, 1)).reshape(B * DIN, S)
    lenrow = jnp.repeat(lengths, DIN).reshape(B * DIN, 1)

    full = lambda shape: pl.BlockSpec(shape, lambda b, *_: (0,) * len(shape))

    stats = pl.pallas_call(
        _stats_kernel,
        grid_spec=pltpu.PrefetchScalarGridSpec(
            num_scalar_prefetch=1,
            grid=(1,),
            in_specs=[full((B * DIN, S)), full((B * DIN, 1))],
            out_specs=full((DIN, 2)),
        ),
        out_shape=jax.ShapeDtypeStruct((DIN, 2), jnp.float32),
        compiler_params=pltpu.CompilerParams(
            dimension_semantics=("arbitrary",)),
    )(lengths, xf, lenrow)

    logits_t, labels_t, scores_t, cat_t = pl.pallas_call(
        _fused_kernel,
        grid_spec=pltpu.PrefetchScalarGridSpec(
            num_scalar_prefetch=1,
            grid=(B,),
            in_specs=[
                pl.BlockSpec((1, DIN, S), lambda b, *_: (b, 0, 0)),
                full((DIN, 2)),
                full((E, DIN)), full((E, 1)),
                full((E, E)), full((E, 1)),
                full((G, E)), full((G, 1)),
                full((C, E + G)), full((C, 1)),
            ],
            out_specs=[
                pl.BlockSpec((1, C, S), lambda b, *_: (b, 0, 0)),
                pl.BlockSpec((1, 1, S), lambda b, *_: (b, 0, 0)),
                pl.BlockSpec((1, 1, S), lambda b, *_: (b, 0, 0)),
                pl.BlockSpec((1, E + G, S), lambda b, *_: (b, 0, 0)),
            ],
        ),
        out_shape=[
            jax.ShapeDtypeStruct((B, C, S), jnp.float32),
            jax.ShapeDtypeStruct((B, 1, S), jnp.int32),
            jax.ShapeDtypeStruct((B, 1, S), jnp.float32),
            jax.ShapeDtypeStruct((B, E + G, S), jnp.float32),
        ],
        compiler_params=pltpu.CompilerParams(
            dimension_semantics=("arbitrary",)),
    )(lengths, xf.reshape(B, DIN, S), stats,
      W1.T, b1.reshape(E, 1), W2.T, b2.reshape(E, 1),
      Wg.T, bg.reshape(G, 1), Wseg.T, bseg.reshape(C, 1))

    logits = jnp.transpose(logits_t, (0, 2, 1))
    labels = jnp.transpose(labels_t, (0, 2, 1))
    scores = jnp.transpose(scores_t, (0, 2, 1))
    cat = jnp.transpose(cat_t, (0, 2, 1))
    return (logits, labels, scores, cat)


# speedup vs baseline: 1.0131x; 1.0131x over previous
import functools

import jax
import jax.numpy as jnp
from jax.experimental import pallas as pl
from jax.experimental.pallas import tpu as pltpu

B, S, DIN, E, G, C = 16, 4096, 8, 64, 128, 8

_dot = functools.partial(jnp.dot, preferred_element_type=jnp.float32)


def _fused_kernel(len_ref, xf_ref, xb_ref, lenrow_ref, w1_ref, b1_ref,
                  w2_ref, b2_ref, wg_ref, bg_ref, wseg_ref, bseg_ref,
                  logits_ref, labels_ref, scores_ref, cat_ref, stats_ref):
    b = pl.program_id(0)
    lane = jax.lax.broadcasted_iota(jnp.int32, (1, S), 1)

    @pl.when(b == 0)
    def _():
        xf = xf_ref[...]
        maskf = lane < lenrow_ref[...]
        xm = jnp.where(maskf, xf, 0.0)
        s1 = jnp.sum(xm, axis=1, keepdims=True)
        s2 = jnp.sum(xm * xm, axis=1, keepdims=True)
        s12 = jnp.concatenate([s1, s2], axis=1)
        sums = s12[0:DIN, :]
        for i in range(1, B):
            sums = sums + s12[i * DIN:(i + 1) * DIN, :]
        cnt = jax.lax.fori_loop(0, B, lambda i, c: c + len_ref[i], 0)
        cntf = jnp.maximum(cnt.astype(jnp.float32), 1.0)
        mean = sums[:, 0:1] / cntf
        var = sums[:, 1:2] / cntf - mean * mean
        scale = 1.0 / jnp.sqrt(var + 1e-6)
        stats_ref[...] = jnp.concatenate([mean, scale], axis=1)

    ln = len_ref[b]
    mask = lane < ln
    xb = xb_ref[0]
    mean = stats_ref[:, 0:1]
    scale = stats_ref[:, 1:2]
    xn = jnp.where(mask, (xb - mean) * scale, 0.0)

    h = jax.nn.relu(_dot(w1_ref[...], xn) + b1_ref[...])
    h = jax.nn.relu(_dot(w2_ref[...], h) + b2_ref[...])
    h = jnp.where(mask, h, 0.0)

    g = jax.nn.relu(_dot(wg_ref[...], h) + bg_ref[...])
    g = jnp.where(mask, g, -jnp.inf)
    gfv = jnp.max(g, axis=1, keepdims=True)

    gfv_bc = jnp.where(mask, gfv, 0.0)
    cat = jnp.concatenate([h, gfv_bc], axis=0)
    cat_ref[0] = cat

    logits = _dot(wseg_ref[...], cat) + bseg_ref[...]
    logits = jnp.where(mask, logits, 0.0)
    logits_ref[0] = logits

    m = jnp.max(logits, axis=0, keepdims=True)
    e = jnp.exp(logits - m)
    p = e / jnp.sum(e, axis=0, keepdims=True)
    score = jnp.max(p, axis=0, keepdims=True)
    cidx = jax.lax.broadcasted_iota(jnp.int32, (C, S), 0)
    lab = jnp.min(jnp.where(p >= score, cidx, C), axis=0, keepdims=True)
    lab = jnp.where(jnp.isnan(score), -1, lab).astype(jnp.int32)
    labels_ref[0] = lab
    scores_ref[0] = score


def kernel(x, lengths, W1, b1, W2, b2, Wg, bg, Wseg, bseg):
    lengths = lengths.astype(jnp.int32)
    xf = jnp.transpose(x, (0, 2, 1)).reshape(B * DIN, S)
    lenrow = jnp.repeat(lengths, DIN).reshape(B * DIN, 1)

    full = lambda shape: pl.BlockSpec(shape, lambda b, *_: (0,) * len(shape))
    logits_t, labels_t, scores_t, cat_t = pl.pallas_call(
        _fused_kernel,
        grid_spec=pltpu.PrefetchScalarGridSpec(
            num_scalar_prefetch=1,
            grid=(B,),
            in_specs=[
                full((B * DIN, S)),
                pl.BlockSpec((1, DIN, S), lambda b, *_: (b, 0, 0)),
                full((B * DIN, 1)),
                full((E, DIN)), full((E, 1)),
                full((E, E)), full((E, 1)),
                full((G, E)), full((G, 1)),
                full((C, E + G)), full((C, 1)),
            ],
            out_specs=[
                pl.BlockSpec((1, C, S), lambda b, *_: (b, 0, 0)),
                pl.BlockSpec((1, 1, S), lambda b, *_: (b, 0, 0)),
                pl.BlockSpec((1, 1, S), lambda b, *_: (b, 0, 0)),
                pl.BlockSpec((1, E + G, S), lambda b, *_: (b, 0, 0)),
            ],
            scratch_shapes=[pltpu.VMEM((DIN, 2), jnp.float32)],
        ),
        out_shape=[
            jax.ShapeDtypeStruct((B, C, S), jnp.float32),
            jax.ShapeDtypeStruct((B, 1, S), jnp.int32),
            jax.ShapeDtypeStruct((B, 1, S), jnp.float32),
            jax.ShapeDtypeStruct((B, E + G, S), jnp.float32),
        ],
        compiler_params=pltpu.CompilerParams(
            dimension_semantics=("arbitrary",)),
    )(lengths, xf, xf.reshape(B, DIN, S), lenrow,
      W1.T, b1.reshape(E, 1), W2.T, b2.reshape(E, 1),
      Wg.T, bg.reshape(G, 1), Wseg.T, bseg.reshape(C, 1))

    logits = jnp.transpose(logits_t, (0, 2, 1))
    labels = jnp.transpose(labels_t, (0, 2, 1))
    scores = jnp.transpose(scores_t, (0, 2, 1))
    cat = jnp.transpose(cat_t, (0, 2, 1))
    return (logits, labels, scores, cat)


# speedup vs baseline: 1.0677x; 1.0539x over previous
import functools

import jax
import jax.numpy as jnp
from jax.experimental import pallas as pl
from jax.experimental.pallas import tpu as pltpu

B, S, DIN, E, G, C = 16, 4096, 8, 64, 128, 8

_dot = functools.partial(jnp.dot, preferred_element_type=jnp.float32)


def _fused_kernel(len_ref, xf_ref, lenrow_ref, w1_ref, b1_ref,
                  w2_ref, b2_ref, wg_ref, bg_ref, wseg_ref, bseg_ref,
                  logits_ref, labels_ref, scores_ref, cat_ref, stats_ref):
    b = pl.program_id(0)
    lane = jax.lax.broadcasted_iota(jnp.int32, (1, S), 1)

    @pl.when(b == 0)
    def _():
        xf = xf_ref[...]
        maskf = lane < lenrow_ref[...]
        xm = jnp.where(maskf, xf, 0.0)
        s1 = jnp.sum(xm, axis=1, keepdims=True)
        s2 = jnp.sum(xm * xm, axis=1, keepdims=True)
        s12 = jnp.concatenate([s1, s2], axis=1)
        sums = s12[0:DIN, :]
        for i in range(1, B):
            sums = sums + s12[i * DIN:(i + 1) * DIN, :]
        cnt = jax.lax.fori_loop(0, B, lambda i, c: c + len_ref[i], 0)
        cntf = jnp.maximum(cnt.astype(jnp.float32), 1.0)
        mean = sums[:, 0:1] / cntf
        var = sums[:, 1:2] / cntf - mean * mean
        scale = 1.0 / jnp.sqrt(var + 1e-6)
        stats_ref[...] = jnp.concatenate([mean, scale], axis=1)

    ln = len_ref[b]
    mask = lane < ln
    xb = xf_ref[pl.ds(b * DIN, DIN), :]
    mean = stats_ref[:, 0:1]
    scale = stats_ref[:, 1:2]
    xn = jnp.where(mask, (xb - mean) * scale, 0.0)

    h = jax.nn.relu(_dot(w1_ref[...], xn) + b1_ref[...])
    h = jax.nn.relu(_dot(w2_ref[...], h) + b2_ref[...])
    h = jnp.where(mask, h, 0.0)

    g = jax.nn.relu(_dot(wg_ref[...], h) + bg_ref[...])
    g = jnp.where(mask, g, -jnp.inf)
    gfv = jnp.max(g, axis=1, keepdims=True)

    gfv_bc = jnp.where(mask, gfv, 0.0)
    cat = jnp.concatenate([h, gfv_bc], axis=0)
    cat_ref[0] = cat

    logits = _dot(wseg_ref[...], cat) + bseg_ref[...]
    logits = jnp.where(mask, logits, 0.0)
    logits_ref[0] = logits

    m = jnp.max(logits, axis=0, keepdims=True)
    e = jnp.exp(logits - m)
    p = e / jnp.sum(e, axis=0, keepdims=True)
    score = jnp.max(p, axis=0, keepdims=True)
    cidx = jax.lax.broadcasted_iota(jnp.int32, (C, S), 0)
    lab = jnp.min(jnp.where(p >= score, cidx, C), axis=0, keepdims=True)
    lab = jnp.where(jnp.isnan(score), -1, lab).astype(jnp.int32)
    labels_ref[0] = lab
    scores_ref[0] = score


def kernel(x, lengths, W1, b1, W2, b2, Wg, bg, Wseg, bseg):
    lengths = lengths.astype(jnp.int32)
    xf = jnp.transpose(x, (0, 2, 1)).reshape(B * DIN, S)
    lenrow = jnp.repeat(lengths, DIN).reshape(B * DIN, 1)

    full = lambda shape: pl.BlockSpec(shape, lambda b, *_: (0,) * len(shape))
    logits_t, labels_t, scores_t, cat_t = pl.pallas_call(
        _fused_kernel,
        grid_spec=pltpu.PrefetchScalarGridSpec(
            num_scalar_prefetch=1,
            grid=(B,),
            in_specs=[
                full((B * DIN, S)),
                full((B * DIN, 1)),
                full((E, DIN)), full((E, 1)),
                full((E, E)), full((E, 1)),
                full((G, E)), full((G, 1)),
                full((C, E + G)), full((C, 1)),
            ],
            out_specs=[
                pl.BlockSpec((1, C, S), lambda b, *_: (b, 0, 0)),
                pl.BlockSpec((1, 1, S), lambda b, *_: (b, 0, 0)),
                pl.BlockSpec((1, 1, S), lambda b, *_: (b, 0, 0)),
                pl.BlockSpec((1, E + G, S), lambda b, *_: (b, 0, 0)),
            ],
            scratch_shapes=[pltpu.VMEM((DIN, 2), jnp.float32)],
        ),
        out_shape=[
            jax.ShapeDtypeStruct((B, C, S), jnp.float32),
            jax.ShapeDtypeStruct((B, 1, S), jnp.int32),
            jax.ShapeDtypeStruct((B, 1, S), jnp.float32),
            jax.ShapeDtypeStruct((B, E + G, S), jnp.float32),
        ],
        compiler_params=pltpu.CompilerParams(
            dimension_semantics=("arbitrary",)),
    )(lengths, xf, lenrow,
      W1.T, b1.reshape(E, 1), W2.T, b2.reshape(E, 1),
      Wg.T, bg.reshape(G, 1), Wseg.T, bseg.reshape(C, 1))

    logits = jnp.transpose(logits_t, (0, 2, 1))
    labels = jnp.transpose(labels_t, (0, 2, 1))
    scores = jnp.transpose(scores_t, (0, 2, 1))
    cat = jnp.transpose(cat_t, (0, 2, 1))
    return (logits, labels, scores, cat)


# speedup vs baseline: 1.3705x; 1.2836x over previous
import functools

import jax
import jax.numpy as jnp
from jax.experimental import pallas as pl
from jax.experimental.pallas import tpu as pltpu

B, S, DIN, E, G, C = 16, 4096, 8, 64, 128, 8

_dott = functools.partial(
    jax.lax.dot_general,
    dimension_numbers=(((0,), (0,)), ((), ())),
    preferred_element_type=jnp.float32,
)


def _fused_kernel(len_ref, xf_ref, lenrow_ref, w1_ref, b1_ref,
                  w2_ref, b2_ref, wg_ref, bg_ref, wseg_ref, bseg_ref,
                  logits_ref, labels_ref, scores_ref, cat_ref, stats_ref):
    b = pl.program_id(0)
    lane = jax.lax.broadcasted_iota(jnp.int32, (1, S), 1)

    @pl.when(b == 0)
    def _():
        xf = xf_ref[...]
        maskf = lane < lenrow_ref[...]
        xm = jnp.where(maskf, xf, 0.0)
        s1 = jnp.sum(xm, axis=1, keepdims=True)
        s2 = jnp.sum(xm * xm, axis=1, keepdims=True)
        s12 = jnp.concatenate([s1, s2], axis=1)
        sums = s12[0:DIN, :]
        for i in range(1, B):
            sums = sums + s12[i * DIN:(i + 1) * DIN, :]
        cnt = jax.lax.fori_loop(0, B, lambda i, c: c + len_ref[i], 0)
        cntf = jnp.maximum(cnt.astype(jnp.float32), 1.0)
        mean = sums[:, 0:1] / cntf
        var = sums[:, 1:2] / cntf - mean * mean
        scale = 1.0 / jnp.sqrt(var + 1e-6)
        stats_ref[...] = jnp.concatenate([mean, scale], axis=1)

    ln = len_ref[b]
    mask = lane < ln
    xb = xf_ref[pl.ds(b * DIN, DIN), :]
    mean = stats_ref[:, 0:1]
    scale = stats_ref[:, 1:2]
    xn = jnp.where(mask, (xb - mean) * scale, 0.0)

    h = jax.nn.relu(_dott(w1_ref[...], xn) + b1_ref[...].reshape(E, 1))
    h = jax.nn.relu(_dott(w2_ref[...], h) + b2_ref[...].reshape(E, 1))
    h = jnp.where(mask, h, 0.0)

    g = jax.nn.relu(_dott(wg_ref[...], h) + bg_ref[...].reshape(G, 1))
    g = jnp.where(mask, g, -jnp.inf)
    gfv = jnp.max(g, axis=1, keepdims=True)

    gfv_bc = jnp.where(mask, gfv, 0.0)
    cat = jnp.concatenate([h, gfv_bc], axis=0)
    cat_ref[0] = cat

    logits = _dott(wseg_ref[...], cat) + bseg_ref[...].reshape(C, 1)
    logits = jnp.where(mask, logits, 0.0)
    logits_ref[0] = logits

    m = jnp.max(logits, axis=0, keepdims=True)
    e = jnp.exp(logits - m)
    p = e / jnp.sum(e, axis=0, keepdims=True)
    score = jnp.max(p, axis=0, keepdims=True)
    cidx = jax.lax.broadcasted_iota(jnp.int32, (C, S), 0)
    lab = jnp.min(jnp.where(p >= score, cidx, C), axis=0, keepdims=True)
    lab = jnp.where(jnp.isnan(score), -1, lab).astype(jnp.int32)
    labels_ref[0] = lab
    scores_ref[0] = score


def kernel(x, lengths, W1, b1, W2, b2, Wg, bg, Wseg, bseg):
    lengths = lengths.astype(jnp.int32)
    xf = jnp.transpose(x, (0, 2, 1)).reshape(B * DIN, S)
    lenrow = jnp.repeat(lengths, DIN).reshape(B * DIN, 1)

    full = lambda shape: pl.BlockSpec(shape, lambda b, *_: (0,) * len(shape))
    logits_t, labels_t, scores_t, cat_t = pl.pallas_call(
        _fused_kernel,
        grid_spec=pltpu.PrefetchScalarGridSpec(
            num_scalar_prefetch=1,
            grid=(B,),
            in_specs=[
                full((B * DIN, S)),
                full((B * DIN, 1)),
                full((DIN, E)), full((1, E)),
                full((E, E)), full((1, E)),
                full((E, G)), full((1, G)),
                full((E + G, C)), full((1, C)),
            ],
            out_specs=[
                pl.BlockSpec((1, C, S), lambda b, *_: (b, 0, 0)),
                pl.BlockSpec((1, 1, S), lambda b, *_: (b, 0, 0)),
                pl.BlockSpec((1, 1, S), lambda b, *_: (b, 0, 0)),
                pl.BlockSpec((1, E + G, S), lambda b, *_: (b, 0, 0)),
            ],
            scratch_shapes=[pltpu.VMEM((DIN, 2), jnp.float32)],
        ),
        out_shape=[
            jax.ShapeDtypeStruct((B, C, S), jnp.float32),
            jax.ShapeDtypeStruct((B, 1, S), jnp.int32),
            jax.ShapeDtypeStruct((B, 1, S), jnp.float32),
            jax.ShapeDtypeStruct((B, E + G, S), jnp.float32),
        ],
        compiler_params=pltpu.CompilerParams(
            dimension_semantics=("arbitrary",)),
    )(lengths, xf, lenrow,
      W1, b1.reshape(1, E), W2, b2.reshape(1, E),
      Wg, bg.reshape(1, G), Wseg, bseg.reshape(1, C))

    logits = jnp.transpose(logits_t, (0, 2, 1))
    labels = jnp.transpose(labels_t, (0, 2, 1))
    scores = jnp.transpose(scores_t, (0, 2, 1))
    cat = jnp.transpose(cat_t, (0, 2, 1))
    return (logits, labels, scores, cat)


# speedup vs baseline: 1.4359x; 1.0477x over previous
import functools

import jax
import jax.numpy as jnp
from jax.experimental import pallas as pl
from jax.experimental.pallas import tpu as pltpu

B, S, DIN, E, G, C = 16, 4096, 8, 64, 128, 8

_dott = functools.partial(
    jax.lax.dot_general,
    dimension_numbers=(((0,), (0,)), ((), ())),
    preferred_element_type=jnp.float32,
)


def _fused_kernel(len_ref, xf_ref, w1_ref, b1_ref,
                  w2_ref, b2_ref, wg_ref, bg_ref, wseg_ref, bseg_ref,
                  logits_ref, labels_ref, scores_ref, cat_ref, stats_ref):
    b = pl.program_id(0)
    lane = jax.lax.broadcasted_iota(jnp.int32, (1, S), 1)

    @pl.when(b == 0)
    def _():
        xf = xf_ref[...]
        rsub = jax.lax.broadcasted_iota(jnp.int32, (B * DIN, 1), 0) // DIN
        lenrow = jnp.zeros((B * DIN, 1), jnp.int32)
        for i in range(B):
            lenrow = jnp.where(rsub == i, len_ref[i], lenrow)
        maskf = lane < lenrow
        xm = jnp.where(maskf, xf, 0.0)
        s1 = jnp.sum(xm, axis=1, keepdims=True)
        s2 = jnp.sum(xm * xm, axis=1, keepdims=True)
        s12 = jnp.concatenate([s1, s2], axis=1)
        sums = s12[0:DIN, :]
        for i in range(1, B):
            sums = sums + s12[i * DIN:(i + 1) * DIN, :]
        cnt = jax.lax.fori_loop(0, B, lambda i, c: c + len_ref[i], 0)
        cntf = jnp.maximum(cnt.astype(jnp.float32), 1.0)
        mean = sums[:, 0:1] / cntf
        var = sums[:, 1:2] / cntf - mean * mean
        scale = 1.0 / jnp.sqrt(var + 1e-6)
        stats_ref[...] = jnp.concatenate([mean, scale], axis=1)

    ln = len_ref[b]
    mask = lane < ln
    xb = xf_ref[pl.ds(b * DIN, DIN), :]
    mean = stats_ref[:, 0:1]
    scale = stats_ref[:, 1:2]
    xn = jnp.where(mask, (xb - mean) * scale, 0.0)

    h = jax.nn.relu(_dott(w1_ref[...], xn) + b1_ref[...].reshape(E, 1))
    h = jax.nn.relu(_dott(w2_ref[...], h) + b2_ref[...].reshape(E, 1))
    h = jnp.where(mask, h, 0.0)

    g = jax.nn.relu(_dott(wg_ref[...], h) + bg_ref[...].reshape(G, 1))
    g = jnp.where(mask, g, -jnp.inf)
    gfv = jnp.max(g, axis=1, keepdims=True)

    gfv_bc = jnp.where(mask, gfv, 0.0)
    cat = jnp.concatenate([h, gfv_bc], axis=0)
    cat_ref[0] = cat

    logits = _dott(wseg_ref[...], cat) + bseg_ref[...].reshape(C, 1)
    logits = jnp.where(mask, logits, 0.0)
    logits_ref[0] = logits

    m = jnp.max(logits, axis=0, keepdims=True)
    e = jnp.exp(logits - m)
    p = e / jnp.sum(e, axis=0, keepdims=True)
    score = jnp.max(p, axis=0, keepdims=True)
    cidx = jax.lax.broadcasted_iota(jnp.int32, (C, S), 0)
    lab = jnp.min(jnp.where(p >= score, cidx, C), axis=0, keepdims=True)
    lab = jnp.where(jnp.isnan(score), -1, lab).astype(jnp.int32)
    labels_ref[0] = lab
    scores_ref[0] = score


def kernel(x, lengths, W1, b1, W2, b2, Wg, bg, Wseg, bseg):
    lengths = lengths.astype(jnp.int32)
    xf = jnp.transpose(x, (0, 2, 1)).reshape(B * DIN, S)

    full = lambda shape: pl.BlockSpec(shape, lambda b, *_: (0,) * len(shape))
    logits_t, labels_t, scores_t, cat_t = pl.pallas_call(
        _fused_kernel,
        grid_spec=pltpu.PrefetchScalarGridSpec(
            num_scalar_prefetch=1,
            grid=(B,),
            in_specs=[
                full((B * DIN, S)),
                full((DIN, E)), full((1, E)),
                full((E, E)), full((1, E)),
                full((E, G)), full((1, G)),
                full((E + G, C)), full((1, C)),
            ],
            out_specs=[
                pl.BlockSpec((1, C, S), lambda b, *_: (b, 0, 0)),
                pl.BlockSpec((1, 1, S), lambda b, *_: (b, 0, 0)),
                pl.BlockSpec((1, 1, S), lambda b, *_: (b, 0, 0)),
                pl.BlockSpec((1, E + G, S), lambda b, *_: (b, 0, 0)),
            ],
            scratch_shapes=[pltpu.VMEM((DIN, 2), jnp.float32)],
        ),
        out_shape=[
            jax.ShapeDtypeStruct((B, C, S), jnp.float32),
            jax.ShapeDtypeStruct((B, 1, S), jnp.int32),
            jax.ShapeDtypeStruct((B, 1, S), jnp.float32),
            jax.ShapeDtypeStruct((B, E + G, S), jnp.float32),
        ],
        compiler_params=pltpu.CompilerParams(
            dimension_semantics=("arbitrary",)),
    )(lengths, xf,
      W1, b1.reshape(1, E), W2, b2.reshape(1, E),
      Wg, bg.reshape(1, G), Wseg, bseg.reshape(1, C))

    logits = jnp.transpose(logits_t, (0, 2, 1))
    labels = jnp.transpose(labels_t, (0, 2, 1))
    scores = jnp.transpose(scores_t, (0, 2, 1))
    cat = jnp.transpose(cat_t, (0, 2, 1))
    return (logits, labels, scores, cat)
